# Initial kernel scaffold; baseline (speedup 1.0000x reference)
import jax
import jax.numpy as jnp
from jax.experimental import pallas as pl


def kernel(stimuli, eye):
    raise NotImplementedError("write your pallas kernel here")


# speedup vs baseline: 3.2114x; 3.2114x over previous
import functools

import jax
import jax.numpy as jnp
from jax import lax
from jax.experimental import pallas as pl
from jax.experimental.pallas import tpu as pltpu
from jax.experimental.pallas import tpu_sc as plsc

H, W = 304, 608
HW = H * W
NF = 32
NC, NS, L = 2, 16, 16
ROWS_PER_CHUNK = 4
CB = ROWS_PER_CHUNK * W
NCHUNK = H // ROWS_PER_CHUNK
NITER = NCHUNK // 2
GPR = W // L

_mesh = plsc.VectorSubcoreMesh(
    core_axis_name="c", subcore_axis_name="s", num_cores=NC, num_subcores=NS)

_IDX4 = [pltpu.VMEM((CB,), jnp.int32)] * 4
_F4 = [pltpu.VMEM((CB,), jnp.float32)] * 4


@functools.partial(
    pl.kernel,
    out_type=jax.ShapeDtypeStruct((NF * HW,), jnp.float32),
    mesh=_mesh,
    scratch_types=[
        pltpu.VMEM((16,), jnp.float32),
        pltpu.VMEM((W,), jnp.float32),
        pltpu.VMEM((H * L,), jnp.float32),
        pltpu.VMEM((H * L,), jnp.float32),
        *_IDX4, *_F4, *_F4,
        *_IDX4, *_F4, *_F4,
        pltpu.VMEM((CB,), jnp.float32),
        pltpu.SemaphoreType.DMA,
        pltpu.SemaphoreType.DMA,
    ],
)
def _warp(im_hbm, consts_hbm, xg_hbm, rx_hbm, ry_hbm, out_hbm,
          cv, xgv, rxv, ryv,
          ia0, ib0, ic0, id0, wa0, wb0, wc0, wd0, ga0, gb0, gc0, gd0,
          ia1, ib1, ic1, id1, wa1, wb1, wc1, wd1, ga1, gb1, gc1, gd1,
          ob, sem0, sem1):
    wid = lax.axis_index("s") * NC + lax.axis_index("c")
    pltpu.sync_copy(consts_hbm.at[wid], cv)
    pltpu.sync_copy(xg_hbm, xgv)
    pltpu.sync_copy(rx_hbm.at[wid], rxv)
    pltpu.sync_copy(ry_hbm.at[wid], ryv)
    cvv = cv[...]
    a00 = jnp.full((L,), cvv[0])
    a02 = jnp.full((L,), cvv[1])
    a10 = jnp.full((L,), cvv[2])
    a12 = jnp.full((L,), cvv[3])
    base = wid * HW

    set0 = (ia0, ib0, ic0, id0, wa0, wb0, wc0, wd0, ga0, gb0, gc0, gd0)
    set1 = (ia1, ib1, ic1, id1, wa1, wb1, wc1, wd1, ga1, gb1, gc1, gd1)

    def compute(c, bufs):
        ia, ib, ic, id_, wa, wb, wc, wd = bufs[:8]

        def row_body(r, rc):
            i = c * ROWS_PER_CHUNK + r
            px = rxv[pl.ds(i * L, L)]
            py = ryv[pl.ds(i * L, L)]

            def grp_body(g, _):
                xv = xgv[pl.ds(g * L, L)]
                xs = (a00 * xv + px) + a02
                ys = (a10 * xv + py) + a12
                xp = (xs + 1.0) * (W / 2)
                yp = (ys + 1.0) * (H / 2)
                xq = jnp.clip(xp, -1e6, 1e6)
                yq = jnp.clip(yp, -1e6, 1e6)
                xt = xq.astype(jnp.int32)
                xtf = xt.astype(jnp.float32)
                neg_x = xq < xtf
                fxl = jnp.where(neg_x, xt - 1, xt)
                fxf = jnp.where(neg_x, xtf - 1.0, xtf)
                yt = yq.astype(jnp.int32)
                ytf = yt.astype(jnp.float32)
                neg_y = yq < ytf
                fyl = jnp.where(neg_y, yt - 1, yt)
                fyf = jnp.where(neg_y, ytf - 1.0, ytf)
                in_x = (xp >= 0.0) & (xp < W - 1.0)
                in_y = (yp >= 0.0) & (yp < H - 1.0)
                zero = jnp.zeros((L,), jnp.float32)
                hl = jnp.where(in_x, fxf + 1.0 - xp, zero)
                hr = jnp.where(in_x, xp - fxf, zero)
                vt = jnp.where(in_y, fyf + 1.0 - yp, zero)
                vb = jnp.where(in_y, yp - fyf, zero)
                x0 = jnp.clip(fxl, 0, W - 2)
                y0 = jnp.clip(fyl, 0, H - 2)
                idx = base + y0 * W + x0
                sl = pl.ds((r * GPR + g) * L, L)
                ia[sl] = idx
                ib[sl] = idx + W
                ic[sl] = idx + 1
                id_[sl] = idx + (W + 1)
                wa[sl] = hl * vt
                wb[sl] = hl * vb
                wc[sl] = hr * vt
                wd[sl] = hr * vb
                return 0

            lax.fori_loop(0, GPR, grp_body, 0)
            return rc

        lax.fori_loop(0, ROWS_PER_CHUNK, row_body, 0)

    def fire(bufs, sem):
        ia, ib, ic, id_ = bufs[:4]
        ga, gb, gc, gd = bufs[8:12]
        pltpu.async_copy(im_hbm.at[ia], ga, sem)
        pltpu.async_copy(im_hbm.at[ib], gb, sem)
        pltpu.async_copy(im_hbm.at[ic], gc, sem)
        pltpu.async_copy(im_hbm.at[id_], gd, sem)

    def drain(bufs, sem):
        dummy = im_hbm.at[pl.ds(0, CB)]
        for g in bufs[8:12]:
            pltpu.make_async_copy(dummy, g, sem).wait()

    def combine_write(c, bufs):
        wa, wb, wc, wd = bufs[4:8]
        ga, gb, gc, gd = bufs[8:12]

        def comb(t, _):
            sl = pl.ds(t * L, L)
            ob[sl] = (wa[sl] * ga[sl] + wb[sl] * gb[sl]
                      + wc[sl] * gc[sl] + wd[sl] * gd[sl])
            return 0

        lax.fori_loop(0, CB // L, comb, 0)
        pltpu.sync_copy(ob, out_hbm.at[pl.ds(base + c * CB, CB)])

    compute(0, set0)
    fire(set0, sem0)

    def body(k, carry):
        a = 2 * k
        b = a + 1
        compute(b, set1)
        drain(set0, sem0)
        combine_write(a, set0)
        fire(set1, sem1)

        @pl.when(k < NITER - 1)
        def _():
            compute(a + 2, set0)

        drain(set1, sem1)
        combine_write(b, set1)

        @pl.when(k < NITER - 1)
        def _():
            fire(set0, sem0)

        return carry

    lax.fori_loop(0, NITER, body, 0)


def _round_bf16(x):
    b = lax.bitcast_convert_type(x, jnp.uint32)
    b = (b + jnp.uint32(0x7FFF) + ((b >> 16) & jnp.uint32(1))) & jnp.uint32(
        0xFFFF0000)
    return lax.bitcast_convert_type(b, jnp.float32)


def kernel(stimuli, eye):
    im = stimuli.reshape(-1).astype(jnp.float32)
    ab = _round_bf16(eye.reshape(NF, 6).astype(jnp.float32))
    xt = jnp.linspace(-1.0, 1.0, W).astype(jnp.float32)
    yt = jnp.linspace(-1.0, 1.0, H).astype(jnp.float32)
    xg = _round_bf16(xt)
    yg = _round_bf16(yt)
    rx = ab[:, 1:2] * yg[None, :]
    ry = ab[:, 4:5] * yg[None, :]
    rx16 = jnp.broadcast_to(rx[:, :, None], (NF, H, L)).reshape(NF, H * L)
    ry16 = jnp.broadcast_to(ry[:, :, None], (NF, H, L)).reshape(NF, H * L)
    consts = jnp.stack([ab[:, 0], ab[:, 2], ab[:, 3], ab[:, 5]], axis=1)
    consts = jnp.pad(consts, ((0, 0), (0, 12)))
    out = _warp(im, consts, xg, rx16, ry16)
    return out.reshape(stimuli.shape)
